# baseline (speedup 1.0000x reference)
import functools

import jax
import jax.numpy as jnp
from jax import lax
from jax.experimental import pallas as pl
from jax.experimental.pallas import tpu as pltpu
from jax.experimental.pallas import tpu_sc as plsc

N = 10000
E = 320000
D = 128
R = 5
NB = 2
HD = 32
B = 128

NP = 10240
TROWS = NP * R

NC, NS = 2, 16
NW = NC * NS
CHUNK = 128
NBUF = 9
NBATCH = 9
NCHUNK = NBUF * NBATCH
EPW = NCHUNK * CHUNK
EP = EPW * NW
EPB = NBUF * CHUNK
TB = EP // EPB
NB_FAST, NB_SLOW = 13, 5
FAST_CID = 0
NB1 = TB // NS
RPT = TROWS // NS
ZCH = RPT // CHUNK

_mesh = plsc.VectorSubcoreMesh(core_axis_name="c", subcore_axis_name="s",
                               num_cores=NC)
_mesh1 = plsc.VectorSubcoreMesh(core_axis_name="c", subcore_axis_name="s",
                                num_cores=1)


def _zero_spmem(zeros_hbm, zbuf, agg_sh, sid, sem):
  pltpu.sync_copy(zeros_hbm, zbuf)
  descs = []
  for c in range(ZCH):
    descs.append(pltpu.async_copy(
        zbuf, agg_sh.at[pl.ds(sid * RPT + c * CHUNK, CHUNK)], sem))
  for d in descs:
    d.wait()


def _edge_count_body(src_hbm, dst_hbm, et_hbm, ones_hbm, zeros_hbm,
                     cnt_hbm, idx2_hbm,
                     srcv, dstv, etv, idx2, ones, agg_sh, sem):
  cid = lax.axis_index("c")
  sid = lax.axis_index("s")
  wid = cid * NS + sid
  base = wid * EPW

  _zero_spmem(zeros_hbm, ones.at[pl.ds(0, CHUNK)], agg_sh, sid, sem)
  pltpu.sync_copy(ones_hbm, ones)

  pltpu.sync_copy(src_hbm.at[pl.ds(base, EPW)], srcv)
  pltpu.sync_copy(dst_hbm.at[pl.ds(base, EPW)], dstv)
  pltpu.sync_copy(et_hbm.at[pl.ds(base, EPW)], etv)

  def _idx(t, carry):
    for u in range((NBUF * CHUNK) // 16):
      sl = pl.ds(t * NBUF * CHUNK + u * 16, 16)
      su = pl.ds(u * 16, 16)
      tt = etv[sl] * NP
      idx2[t, 0, su] = srcv[sl] + tt
      idx2[t, 1, su] = dstv[sl] + tt
    return carry
  lax.fori_loop(0, NBATCH, _idx, 0)

  pltpu.sync_copy(idx2, idx2_hbm.at[pl.ds(wid * NBATCH, NBATCH)])
  plsc.subcore_barrier()

  for t0 in range(NBATCH // 9 + 1):
    descs = []
    for t in range(t0 * 9, min((t0 + 1) * 9, NBATCH)):
      descs.append(pltpu.async_copy(ones, agg_sh.at[idx2.at[t, 1]], sem,
                                    add=True))
    for d in descs:
      d.wait()
  plsc.subcore_barrier()

  pltpu.sync_copy(agg_sh.at[pl.ds(sid * RPT, RPT)],
                  cnt_hbm.at[cid, pl.ds(sid * RPT, RPT)])


@functools.partial(
    pl.kernel, mesh=_mesh,
    compiler_params=pltpu.CompilerParams(use_tc_tiling_on_sc=False),
    out_type=[
        jax.ShapeDtypeStruct((NC, TROWS, 8), jnp.float32),
        jax.ShapeDtypeStruct((TB, 2, NBUF * CHUNK), jnp.int32),
    ],
    scratch_types=[
        pltpu.VMEM((EPW,), jnp.int32),
        pltpu.VMEM((EPW,), jnp.int32),
        pltpu.VMEM((EPW,), jnp.int32),
        pltpu.VMEM((NBATCH, 2, NBUF * CHUNK), jnp.int32),
        pltpu.VMEM((NBUF * CHUNK, 8), jnp.float32),
        pltpu.VMEM_SHARED((TROWS, 8), jnp.float32),
        pltpu.SemaphoreType.DMA,
    ],
)
def _edge_count(*refs):
  _edge_count_body(*refs)


def _edge_scatter_body(idx2_hbm, hr_hbm, zeros_hbm, out_hbm,
                       idxg, msg, agg_sh, isem, gsem, ssem):
  cid = lax.axis_index("c")
  sid = lax.axis_index("s")

  _zero_spmem(zeros_hbm, msg.at[0, pl.ds(0, CHUNK)], agg_sh, sid, gsem)
  plsc.subcore_barrier()

  def _pipeline(nb, wstart):
    def _load_idx(t):
      return pltpu.async_copy(idx2_hbm.at[wstart + t], idxg.at[t % 2], isem)

    def _fire_gather(t):
      g = t % 2
      return pltpu.async_copy(hr_hbm.at[idxg.at[g, 0]], msg.at[g], gsem)

    def _fire_scatter(t):
      g = t % 2
      return pltpu.async_copy(msg.at[g], agg_sh.at[idxg.at[g, 1]],
                              ssem, add=True)

    _load_idx(0).wait()
    gd = _fire_gather(0)
    sd_prev = None
    for t in range(nb):
      if sd_prev is not None:
        sd_prev.wait()
        sd_prev = None
      idesc = _load_idx(t + 1) if t + 1 < nb else None
      gd.wait()
      if idesc is not None:
        idesc.wait()
        gd = _fire_gather(t + 1)
      sd_prev = _fire_scatter(t)
    sd_prev.wait()

  _pipeline(NB1, sid * NB1)
  plsc.subcore_barrier()

  pltpu.sync_copy(agg_sh.at[pl.ds(sid * RPT, RPT)],
                  out_hbm.at[cid, pl.ds(sid * RPT, RPT)])


@functools.partial(
    pl.kernel, mesh=_mesh1,
    compiler_params=pltpu.CompilerParams(use_tc_tiling_on_sc=False),
    out_type=jax.ShapeDtypeStruct((1, TROWS, HD), jnp.bfloat16),
    scratch_types=[
        pltpu.VMEM((2, 2, NBUF * CHUNK), jnp.int32),
        pltpu.VMEM((2, NBUF * CHUNK, HD), jnp.bfloat16),
        pltpu.VMEM_SHARED((TROWS, HD), jnp.bfloat16),
        pltpu.SemaphoreType.DMA,
        pltpu.SemaphoreType.DMA,
        pltpu.SemaphoreType.DMA,
    ],
)
def _edge_scatter(*refs):
  _edge_scatter_body(*refs)


def _transform_body(comp_ref, h_ref, bases_ref, root_ref, hr_ref, hroot_ref):
  h = h_ref[...]
  hb0 = jnp.dot(h, bases_ref[0], preferred_element_type=jnp.float32)
  hb1 = jnp.dot(h, bases_ref[1], preferred_element_type=jnp.float32)
  for r in range(R):
    hr_ref[r] = (comp_ref[r, 0] * hb0
                 + comp_ref[r, 1] * hb1).astype(jnp.bfloat16)
  hroot_ref[...] = jnp.dot(h, root_ref[...], preferred_element_type=jnp.float32)


def _transform(h, bases, comp, root, bn=1024):
  din = h.shape[1]
  return pl.pallas_call(
      _transform_body,
      grid=(NP // bn,),
      in_specs=[
          pl.BlockSpec(memory_space=pltpu.SMEM),
          pl.BlockSpec((bn, din), lambda i: (i, 0)),
          pl.BlockSpec((NB, din, HD), lambda i: (0, 0, 0)),
          pl.BlockSpec((din, HD), lambda i: (0, 0)),
      ],
      out_specs=[
          pl.BlockSpec((R, bn, HD), lambda i: (0, i, 0)),
          pl.BlockSpec((bn, HD), lambda i: (i, 0)),
      ],
      out_shape=[
          jax.ShapeDtypeStruct((R, NP, HD), jnp.bfloat16),
          jax.ShapeDtypeStruct((NP, HD), jnp.float32),
      ],
  )(comp, h, bases, root)


def _combine_body(agg_ref, cnt_ref, hroot_ref, bias_ref, out_ref):
  a = agg_ref[0]
  c = cnt_ref[0]
  for k in range(1, NC):
    c = c + cnt_ref[k]
  acc = hroot_ref[...] + bias_ref[...]
  for r in range(R):
    inv = 1.0 / jnp.maximum(c[r, :, 0:1], 1.0)
    acc = acc + a[r].astype(jnp.float32) * inv
  out_ref[...] = jnp.tanh(acc)


def _combine(agg, cnt, hroot, bias, bn=2048):
  return pl.pallas_call(
      _combine_body,
      grid=(NP // bn,),
      in_specs=[
          pl.BlockSpec((1, R, bn, HD), lambda i: (0, 0, i, 0)),
          pl.BlockSpec((NC, R, bn, 8), lambda i: (0, 0, i, 0)),
          pl.BlockSpec((bn, HD), lambda i: (i, 0)),
          pl.BlockSpec((1, HD), lambda i: (0, 0)),
      ],
      out_specs=pl.BlockSpec((bn, HD), lambda i: (i, 0)),
      out_shape=jax.ShapeDtypeStruct((NP, HD), jnp.float32),
  )(agg, cnt, hroot, bias)


def _fused_body(comp_ref, agg_ref, cnt_ref, hroot_ref, bias_ref,
                bases_ref, root_ref, h_ref, hr_ref, hroot2_ref):
  a = agg_ref[0]
  c = cnt_ref[0]
  for k in range(1, NC):
    c = c + cnt_ref[k]
  acc = hroot_ref[...] + bias_ref[...]
  for r in range(R):
    inv = 1.0 / jnp.maximum(c[r, :, 0:1], 1.0)
    acc = acc + a[r].astype(jnp.float32) * inv
  h = jnp.tanh(acc)
  h_ref[...] = h
  hb0 = jnp.dot(h, bases_ref[0], preferred_element_type=jnp.float32)
  hb1 = jnp.dot(h, bases_ref[1], preferred_element_type=jnp.float32)
  for r in range(R):
    hr_ref[r] = (comp_ref[r, 0] * hb0
                 + comp_ref[r, 1] * hb1).astype(jnp.bfloat16)
  hroot2_ref[...] = jnp.dot(h, root_ref[...],
                            preferred_element_type=jnp.float32)


def _fused(agg, cnt, hroot, bias, bases, comp, root, bn=2048):
  return pl.pallas_call(
      _fused_body,
      grid=(NP // bn,),
      in_specs=[
          pl.BlockSpec(memory_space=pltpu.SMEM),
          pl.BlockSpec((1, R, bn, HD), lambda i: (0, 0, i, 0)),
          pl.BlockSpec((NC, R, bn, 8), lambda i: (0, 0, i, 0)),
          pl.BlockSpec((bn, HD), lambda i: (i, 0)),
          pl.BlockSpec((1, HD), lambda i: (0, 0)),
          pl.BlockSpec((NB, HD, HD), lambda i: (0, 0, 0)),
          pl.BlockSpec((HD, HD), lambda i: (0, 0)),
      ],
      out_specs=[
          pl.BlockSpec((bn, HD), lambda i: (i, 0)),
          pl.BlockSpec((R, bn, HD), lambda i: (0, i, 0)),
          pl.BlockSpec((bn, HD), lambda i: (i, 0)),
      ],
      out_shape=[
          jax.ShapeDtypeStruct((NP, HD), jnp.float32),
          jax.ShapeDtypeStruct((R, NP, HD), jnp.bfloat16),
          jax.ShapeDtypeStruct((NP, HD), jnp.float32),
      ],
  )(comp, agg, cnt, hroot, bias, bases, root)


def _head_body(feat_ref, w1_ref, b1_ref, w2_ref, b2_ref, out_ref):
  h1 = jnp.dot(feat_ref[...], w1_ref[...], preferred_element_type=jnp.float32)
  h1 = jax.nn.relu(h1 + b1_ref[...])
  logits = jnp.dot(h1, w2_ref[...], preferred_element_type=jnp.float32)
  logits = logits + b2_ref[...]
  col = lax.broadcasted_iota(jnp.int32, logits.shape, 1)
  logits = jnp.where(col < R, logits, -1e30)
  m = jnp.max(logits, axis=1, keepdims=True)
  lse = jnp.log(jnp.sum(jnp.exp(logits - m), axis=1, keepdims=True)) + m
  out_ref[...] = logits - lse


def _head(feat, w1, b1, w2p, b2p):
  return pl.pallas_call(
      _head_body,
      out_shape=jax.ShapeDtypeStruct((B, 128), jnp.float32),
  )(feat, w1, b1, w2p, b2p)


def kernel(x, edge_index, edge_type, params):
  src = edge_index[0]
  dst = edge_index[1]
  et = edge_type

  pad = EP - E
  src_p = jnp.concatenate([src, jnp.zeros((pad,), jnp.int32)])
  dst_p = jnp.concatenate([dst, jnp.full((pad,), N, jnp.int32)])
  et_p = jnp.concatenate([et, jnp.zeros((pad,), jnp.int32)])

  x_p = jnp.pad(x, ((0, NP - N), (0, 0)))

  ones8 = jnp.ones((NBUF * CHUNK, 8), jnp.float32)
  zeros8 = jnp.zeros((CHUNK, 8), jnp.float32)
  zeros32 = jnp.zeros((CHUNK, HD), jnp.bfloat16)

  cnt, idx2 = _edge_count(src_p, dst_p, et_p, ones8, zeros8)
  cnt2 = cnt.reshape(NC, R, NP, 8)

  cv = params['convs']
  hr, hroot = _transform(x_p, cv[0]['bases'], cv[0]['comp'], cv[0]['root'])
  states = []
  for l in range(4):
    agg = _edge_scatter(idx2, hr.reshape(TROWS, HD), zeros32)
    agg = agg.reshape(1, R, NP, HD)
    bias = cv[l]['bias'].reshape(1, HD)
    if l < 3:
      nxt = cv[l + 1]
      h, hr, hroot = _fused(agg, cnt2, hroot, bias,
                            nxt['bases'], nxt['comp'], nxt['root'])
    else:
      h = _combine(agg, cnt2, hroot, bias)
    states.append(h)

  user = jnp.concatenate([s[:B] for s in states], axis=1)
  item = jnp.concatenate([s[B:2 * B] for s in states], axis=1)
  feat = jnp.concatenate([user, item], axis=1)

  w2p = jnp.pad(params['lin2_w'], ((0, 0), (0, 128 - R)))
  b2p = jnp.pad(params['lin2_b'], (0, 128 - R)).reshape(1, 128)
  out = _head(feat, params['lin1_w'], params['lin1_b'].reshape(1, 128),
              w2p, b2p)
  return out[:, :R]


# speedup vs baseline: 1.1257x; 1.1257x over previous
import functools

import jax
import jax.numpy as jnp
from jax import lax
from jax.experimental import pallas as pl
from jax.experimental.pallas import tpu as pltpu
from jax.experimental.pallas import tpu_sc as plsc

N = 10000
E = 320000
D = 128
R = 5
NB = 2
HD = 32
B = 128

NP = 10240
TROWS = NP * R

NC, NS = 2, 16
NW = NC * NS
CHUNK = 128
NBUF = 9
NBATCH = 9
NCHUNK = NBUF * NBATCH
EPW = NCHUNK * CHUNK
EP = EPW * NW
EPB = NBUF * CHUNK
TB = EP // EPB
NB_FAST, NB_SLOW = 13, 5
FAST_CID = 0
NB1 = TB // NS
RPT = TROWS // NS
ZCH = RPT // CHUNK

_mesh = plsc.VectorSubcoreMesh(core_axis_name="c", subcore_axis_name="s",
                               num_cores=NC)
_mesh1 = plsc.VectorSubcoreMesh(core_axis_name="c", subcore_axis_name="s",
                                num_cores=1)


def _zero_spmem(zeros_hbm, zbuf, agg_sh, sid, sem):
  pltpu.sync_copy(zeros_hbm, zbuf)
  descs = []
  for c in range(ZCH):
    descs.append(pltpu.async_copy(
        zbuf, agg_sh.at[pl.ds(sid * RPT + c * CHUNK, CHUNK)], sem))
  for d in descs:
    d.wait()


def _edge_count_body(src_hbm, dst_hbm, et_hbm, ones_hbm, zeros_hbm,
                     cnt_hbm, idx2_hbm,
                     srcv, dstv, etv, idx2, ones, agg_sh, sem):
  cid = lax.axis_index("c")
  sid = lax.axis_index("s")
  wid = cid * NS + sid
  base = wid * EPW

  _zero_spmem(zeros_hbm, ones.at[pl.ds(0, CHUNK)], agg_sh, sid, sem)
  pltpu.sync_copy(ones_hbm, ones)

  pltpu.sync_copy(src_hbm.at[pl.ds(base, EPW)], srcv)
  pltpu.sync_copy(dst_hbm.at[pl.ds(base, EPW)], dstv)
  pltpu.sync_copy(et_hbm.at[pl.ds(base, EPW)], etv)

  def _idx(t, carry):
    for u in range((NBUF * CHUNK) // 16):
      sl = pl.ds(t * NBUF * CHUNK + u * 16, 16)
      su = pl.ds(u * 16, 16)
      tt = etv[sl]
      idx2[t, 0, su] = srcv[sl] * R + tt
      idx2[t, 1, su] = dstv[sl] * R + tt
    return carry
  lax.fori_loop(0, NBATCH, _idx, 0)

  pltpu.sync_copy(idx2, idx2_hbm.at[pl.ds(wid * NBATCH, NBATCH)])
  plsc.subcore_barrier()

  for t0 in range(NBATCH // 9 + 1):
    descs = []
    for t in range(t0 * 9, min((t0 + 1) * 9, NBATCH)):
      descs.append(pltpu.async_copy(ones, agg_sh.at[idx2.at[t, 1]], sem,
                                    add=True))
    for d in descs:
      d.wait()
  plsc.subcore_barrier()

  pltpu.sync_copy(agg_sh.at[pl.ds(sid * RPT, RPT)],
                  cnt_hbm.at[cid, pl.ds(sid * RPT, RPT)])


@functools.partial(
    pl.kernel, mesh=_mesh,
    compiler_params=pltpu.CompilerParams(use_tc_tiling_on_sc=False),
    out_type=[
        jax.ShapeDtypeStruct((NC, TROWS, 8), jnp.float32),
        jax.ShapeDtypeStruct((TB, 2, NBUF * CHUNK), jnp.int32),
    ],
    scratch_types=[
        pltpu.VMEM((EPW,), jnp.int32),
        pltpu.VMEM((EPW,), jnp.int32),
        pltpu.VMEM((EPW,), jnp.int32),
        pltpu.VMEM((NBATCH, 2, NBUF * CHUNK), jnp.int32),
        pltpu.VMEM((NBUF * CHUNK, 8), jnp.float32),
        pltpu.VMEM_SHARED((TROWS, 8), jnp.float32),
        pltpu.SemaphoreType.DMA,
    ],
)
def _edge_count(*refs):
  _edge_count_body(*refs)


def _edge_scatter_body(idx2_hbm, hr_hbm, zeros_hbm, out_hbm,
                       idxg, msg, agg_sh, isem, gsem, ssem):
  cid = lax.axis_index("c")
  sid = lax.axis_index("s")

  _zero_spmem(zeros_hbm, msg.at[0, pl.ds(0, CHUNK)], agg_sh, sid, gsem)
  plsc.subcore_barrier()

  def _pipeline(nb, wstart):
    def _load_idx(t):
      return pltpu.async_copy(idx2_hbm.at[wstart + t], idxg.at[t % 2], isem)

    def _fire_gather(t):
      g = t % 2
      return pltpu.async_copy(hr_hbm.at[idxg.at[g, 0]], msg.at[g], gsem)

    def _fire_scatter(t):
      g = t % 2
      return pltpu.async_copy(msg.at[g], agg_sh.at[idxg.at[g, 1]],
                              ssem, add=True)

    _load_idx(0).wait()
    gd = _fire_gather(0)
    sd_prev = None
    for t in range(nb):
      if sd_prev is not None:
        sd_prev.wait()
        sd_prev = None
      idesc = _load_idx(t + 1) if t + 1 < nb else None
      gd.wait()
      if idesc is not None:
        idesc.wait()
        gd = _fire_gather(t + 1)
      sd_prev = _fire_scatter(t)
    sd_prev.wait()

  _pipeline(NB1, sid * NB1)
  plsc.subcore_barrier()

  pltpu.sync_copy(agg_sh.at[pl.ds(sid * RPT, RPT)],
                  out_hbm.at[cid, pl.ds(sid * RPT, RPT)])


@functools.partial(
    pl.kernel, mesh=_mesh1,
    compiler_params=pltpu.CompilerParams(use_tc_tiling_on_sc=False),
    out_type=jax.ShapeDtypeStruct((1, TROWS, HD), jnp.bfloat16),
    scratch_types=[
        pltpu.VMEM((2, 2, NBUF * CHUNK), jnp.int32),
        pltpu.VMEM((2, NBUF * CHUNK, HD), jnp.bfloat16),
        pltpu.VMEM_SHARED((TROWS, HD), jnp.bfloat16),
        pltpu.SemaphoreType.DMA,
        pltpu.SemaphoreType.DMA,
        pltpu.SemaphoreType.DMA,
    ],
)
def _edge_scatter(*refs):
  _edge_scatter_body(*refs)


def _transform_body(comp_ref, h_ref, bases_ref, root_ref, hr_ref, hroot_ref):
  h = h_ref[...]
  hb0 = jnp.dot(h, bases_ref[0], preferred_element_type=jnp.float32)
  hb1 = jnp.dot(h, bases_ref[1], preferred_element_type=jnp.float32)
  for r in range(R):
    hr_ref[:, r * HD:(r + 1) * HD] = (
        comp_ref[r, 0] * hb0 + comp_ref[r, 1] * hb1).astype(jnp.bfloat16)
  hroot_ref[...] = jnp.dot(h, root_ref[...], preferred_element_type=jnp.float32)


def _transform(h, bases, comp, root, bn=1024):
  din = h.shape[1]
  return pl.pallas_call(
      _transform_body,
      grid=(NP // bn,),
      in_specs=[
          pl.BlockSpec(memory_space=pltpu.SMEM),
          pl.BlockSpec((bn, din), lambda i: (i, 0)),
          pl.BlockSpec((NB, din, HD), lambda i: (0, 0, 0)),
          pl.BlockSpec((din, HD), lambda i: (0, 0)),
      ],
      out_specs=[
          pl.BlockSpec((bn, R * HD), lambda i: (i, 0)),
          pl.BlockSpec((bn, HD), lambda i: (i, 0)),
      ],
      out_shape=[
          jax.ShapeDtypeStruct((NP, R * HD), jnp.bfloat16),
          jax.ShapeDtypeStruct((NP, HD), jnp.float32),
      ],
  )(comp, h, bases, root)


def _combine_body(agg_ref, cnt_ref, hroot_ref, bias_ref, out_ref):
  a = agg_ref[0].astype(jnp.float32)
  c = cnt_ref[0]
  for k in range(1, NC):
    c = c + cnt_ref[k]
  acc = hroot_ref[...] + bias_ref[...]
  for r in range(R):
    inv = 1.0 / jnp.maximum(c[:, r * 8:r * 8 + 1], 1.0)
    acc = acc + a[:, r * HD:(r + 1) * HD] * inv
  out_ref[...] = jnp.tanh(acc)


def _combine(agg, cnt, hroot, bias, bn=2048):
  return pl.pallas_call(
      _combine_body,
      grid=(NP // bn,),
      in_specs=[
          pl.BlockSpec((1, bn, R * HD), lambda i: (0, i, 0)),
          pl.BlockSpec((NC, bn, R * 8), lambda i: (0, i, 0)),
          pl.BlockSpec((bn, HD), lambda i: (i, 0)),
          pl.BlockSpec((1, HD), lambda i: (0, 0)),
      ],
      out_specs=pl.BlockSpec((bn, HD), lambda i: (i, 0)),
      out_shape=jax.ShapeDtypeStruct((NP, HD), jnp.float32),
  )(agg, cnt, hroot, bias)


def _fused_body(comp_ref, agg_ref, cnt_ref, hroot_ref, bias_ref,
                bases_ref, root_ref, h_ref, hr_ref, hroot2_ref):
  a = agg_ref[0].astype(jnp.float32)
  c = cnt_ref[0]
  for k in range(1, NC):
    c = c + cnt_ref[k]
  acc = hroot_ref[...] + bias_ref[...]
  for r in range(R):
    inv = 1.0 / jnp.maximum(c[:, r * 8:r * 8 + 1], 1.0)
    acc = acc + a[:, r * HD:(r + 1) * HD] * inv
  h = jnp.tanh(acc)
  h_ref[...] = h
  hb0 = jnp.dot(h, bases_ref[0], preferred_element_type=jnp.float32)
  hb1 = jnp.dot(h, bases_ref[1], preferred_element_type=jnp.float32)
  for r in range(R):
    hr_ref[:, r * HD:(r + 1) * HD] = (
        comp_ref[r, 0] * hb0 + comp_ref[r, 1] * hb1).astype(jnp.bfloat16)
  hroot2_ref[...] = jnp.dot(h, root_ref[...],
                            preferred_element_type=jnp.float32)


def _fused(agg, cnt, hroot, bias, bases, comp, root, bn=2048):
  return pl.pallas_call(
      _fused_body,
      grid=(NP // bn,),
      in_specs=[
          pl.BlockSpec(memory_space=pltpu.SMEM),
          pl.BlockSpec((1, bn, R * HD), lambda i: (0, i, 0)),
          pl.BlockSpec((NC, bn, R * 8), lambda i: (0, i, 0)),
          pl.BlockSpec((bn, HD), lambda i: (i, 0)),
          pl.BlockSpec((1, HD), lambda i: (0, 0)),
          pl.BlockSpec((NB, HD, HD), lambda i: (0, 0, 0)),
          pl.BlockSpec((HD, HD), lambda i: (0, 0)),
      ],
      out_specs=[
          pl.BlockSpec((bn, HD), lambda i: (i, 0)),
          pl.BlockSpec((bn, R * HD), lambda i: (i, 0)),
          pl.BlockSpec((bn, HD), lambda i: (i, 0)),
      ],
      out_shape=[
          jax.ShapeDtypeStruct((NP, HD), jnp.float32),
          jax.ShapeDtypeStruct((NP, R * HD), jnp.bfloat16),
          jax.ShapeDtypeStruct((NP, HD), jnp.float32),
      ],
  )(comp, agg, cnt, hroot, bias, bases, root)


def _head_body(feat_ref, w1_ref, b1_ref, w2_ref, b2_ref, out_ref):
  h1 = jnp.dot(feat_ref[...], w1_ref[...], preferred_element_type=jnp.float32)
  h1 = jax.nn.relu(h1 + b1_ref[...])
  logits = jnp.dot(h1, w2_ref[...], preferred_element_type=jnp.float32)
  logits = logits + b2_ref[...]
  col = lax.broadcasted_iota(jnp.int32, logits.shape, 1)
  logits = jnp.where(col < R, logits, -1e30)
  m = jnp.max(logits, axis=1, keepdims=True)
  lse = jnp.log(jnp.sum(jnp.exp(logits - m), axis=1, keepdims=True)) + m
  out_ref[...] = logits - lse


def _head(feat, w1, b1, w2p, b2p):
  return pl.pallas_call(
      _head_body,
      out_shape=jax.ShapeDtypeStruct((B, 128), jnp.float32),
  )(feat, w1, b1, w2p, b2p)


def kernel(x, edge_index, edge_type, params):
  src = edge_index[0]
  dst = edge_index[1]
  et = edge_type

  pad = EP - E
  src_p = jnp.concatenate([src, jnp.zeros((pad,), jnp.int32)])
  dst_p = jnp.concatenate([dst, jnp.full((pad,), N, jnp.int32)])
  et_p = jnp.concatenate([et, jnp.zeros((pad,), jnp.int32)])

  x_p = jnp.pad(x, ((0, NP - N), (0, 0)))

  ones8 = jnp.ones((NBUF * CHUNK, 8), jnp.float32)
  zeros8 = jnp.zeros((CHUNK, 8), jnp.float32)
  zeros32 = jnp.zeros((CHUNK, HD), jnp.bfloat16)

  cnt, idx2 = _edge_count(src_p, dst_p, et_p, ones8, zeros8)
  cnt2 = cnt.reshape(NC, NP, R * 8)

  cv = params['convs']
  hr, hroot = _transform(x_p, cv[0]['bases'], cv[0]['comp'], cv[0]['root'])
  states = []
  for l in range(4):
    agg = _edge_scatter(idx2, hr.reshape(TROWS, HD), zeros32)
    agg = agg.reshape(1, NP, R * HD)
    bias = cv[l]['bias'].reshape(1, HD)
    if l < 3:
      nxt = cv[l + 1]
      h, hr, hroot = _fused(agg, cnt2, hroot, bias,
                            nxt['bases'], nxt['comp'], nxt['root'])
    else:
      h = _combine(agg, cnt2, hroot, bias)
    states.append(h)

  user = jnp.concatenate([s[:B] for s in states], axis=1)
  item = jnp.concatenate([s[B:2 * B] for s in states], axis=1)
  feat = jnp.concatenate([user, item], axis=1)

  w2p = jnp.pad(params['lin2_w'], ((0, 0), (0, 128 - R)))
  b2p = jnp.pad(params['lin2_b'], (0, 128 - R)).reshape(1, 128)
  out = _head(feat, params['lin1_w'], params['lin1_b'].reshape(1, 128),
              w2p, b2p)
  return out[:, :R]
